# Initial kernel scaffold; baseline (speedup 1.0000x reference)
import jax
import jax.numpy as jnp
from jax.experimental import pallas as pl


def kernel(x, thresholds, table1, table2, action_scale, action_bias):
    raise NotImplementedError("write your pallas kernel here")


# speedup vs baseline: 1.2757x; 1.2757x over previous
import functools

import jax
import jax.numpy as jnp
from jax import lax
from jax.experimental import pallas as pl
from jax.experimental.pallas import tpu as pltpu
from jax.experimental.pallas import tpu_sc as plsc

B = 4096
OBS = 64
BITS = 12
NK = BITS + 1
SIZE = 512
NR2 = 43
NV = 4096

NC = 2
NS = 16
NW = NC * NS
BPW = B // NW
IPW = BPW * NR2
CHUNK = 128
NCHUNK = IPW // CHUNK


NBB = 4
BB = B // NBB


def _addr_body(x_ref, thr_ref, tab_ref, out_ref, w_ref):
    k = pl.program_id(1)
    w_ref[k, :, :] = tab_ref[:, 0, 0, :]

    @pl.when(k == NK - 1)
    def _():
        x = x_ref[...]
        thr = thr_ref[...]
        c = jnp.zeros((BB, OBS), jnp.int32)
        for t in range(BITS):
            c = c + (x > thr[:, t][None, :]).astype(jnp.int32)
        e = jnp.concatenate(
            [(c == kk).astype(jnp.float32) for kk in range(NK)], axis=1)
        w = w_ref[...].reshape(NK * OBS, SIZE)
        h1 = jnp.dot(e, w, precision=lax.Precision.HIGHEST,
                     preferred_element_type=jnp.float32)
        bits = (h1 > 0).astype(jnp.float32)
        ri = lax.broadcasted_iota(jnp.int32, (SIZE, NR2), 0)
        cj = lax.broadcasted_iota(jnp.int32, (SIZE, NR2), 1)
        pw = jnp.where(ri // BITS == cj,
                       jnp.left_shift(1, ri % BITS), 0).astype(jnp.float32)
        af = jnp.dot(bits, pw, precision=lax.Precision.HIGHEST,
                     preferred_element_type=jnp.float32)
        ramj = lax.broadcasted_iota(jnp.int32, (BB, NR2), 1)
        out_ref[...] = af.astype(jnp.int32) + NV * ramj


def _layer2_addresses(x, thresholds, table1):
    return pl.pallas_call(
        _addr_body,
        grid=(NBB, NK),
        in_specs=[
            pl.BlockSpec((BB, OBS), lambda b, k: (b, 0)),
            pl.BlockSpec((OBS, BITS), lambda b, k: (0, 0)),
            pl.BlockSpec((OBS, 1, 1, SIZE),
                         lambda b, k: (0, lax.shift_left(1, k) - 1, 0, 0)),
        ],
        out_specs=pl.BlockSpec((BB, NR2), lambda b, k: (b, 0)),
        out_shape=jax.ShapeDtypeStruct((B, NR2), jnp.int32),
        scratch_shapes=[pltpu.VMEM((NK, OBS, SIZE), jnp.float32)],
    )(x, thresholds, table1.reshape(OBS, NV, 1, SIZE))


def _sc_body(idx_hbm, tab_hbm, sb_hbm, out_hbm,
             idx_v, rows_v, out_v, sb_v, sem):
    wid = lax.axis_index("s") * NC + lax.axis_index("c")
    pltpu.sync_copy(idx_hbm.at[pl.ds(wid * IPW, IPW)], idx_v)
    pltpu.sync_copy(sb_hbm, sb_v)

    groups = [list(range(g, min(g + 11, NCHUNK))) for g in range(0, NCHUNK, 11)]
    copies = {}

    def fire(g):
        for c in groups[g]:
            copies[c] = pltpu.async_copy(
                tab_hbm.at[idx_v.at[pl.ds(c * CHUNK, CHUNK)]],
                rows_v.at[pl.ds(c * CHUNK, CHUNK), :], sem)

    def drain(g):
        for c in groups[g]:
            copies[c].wait()

    fire(0)
    for g in range(1, len(groups)):
        fire(g)
        drain(g - 1)
    drain(len(groups) - 1)

    iota = lax.iota(jnp.int32, 16)
    lo = jnp.bitwise_and(iota, 7)
    scale16 = sb_v[pl.ds(0, 16)]
    bias16 = sb_v[pl.ds(16, 16)]

    def reduce_one(bl):
        acc = jnp.zeros((16,), jnp.float32)
        for j in range(NR2):
            acc = acc + rows_v[bl * NR2 + j, :]
        return acc

    def pair(p, _):
        acc_a = reduce_one(2 * p)
        acc_b = reduce_one(2 * p + 1)
        perm_b = lax.gather(
            acc_b, lo[:, None],
            lax.GatherDimensionNumbers(offset_dims=(), collapsed_slice_dims=(0,),
                                       start_index_map=(0,)),
            slice_sizes=(1,), mode=lax.GatherScatterMode.PROMISE_IN_BOUNDS)
        acc = jnp.where(iota < 8, acc_a, perm_b)
        e = jnp.exp(acc + acc)
        th = 1.0 - 2.0 / (e + 1.0)
        out_v[pl.ds(p * 16, 16)] = th * scale16 + bias16
        return _

    lax.fori_loop(0, BPW // 2, pair, None)
    pltpu.sync_copy(out_v, out_hbm.at[pl.ds(wid * BPW * 8, BPW * 8)])


def _sc_gather(flat_idx, tab2, sb):
    call = functools.partial(
        pl.kernel,
        out_type=jax.ShapeDtypeStruct((B * 8,), jnp.float32),
        mesh=plsc.VectorSubcoreMesh(
            core_axis_name="c", subcore_axis_name="s", num_cores=NC),
        compiler_params=pltpu.CompilerParams(use_tc_tiling_on_sc=False),
        scratch_types=[
            pltpu.VMEM((IPW,), jnp.int32),
            pltpu.VMEM((IPW, 16), jnp.float32),
            pltpu.VMEM((BPW * 8,), jnp.float32),
            pltpu.VMEM((32,), jnp.float32),
            pltpu.SemaphoreType.DMA,
        ],
    )(_sc_body)
    return call(flat_idx, tab2, sb)


def kernel(x, thresholds, table1, table2, action_scale, action_bias):
    flat_idx = _layer2_addresses(x, thresholds, table1).reshape(-1)
    tab2 = jnp.pad(table2.reshape(NR2 * NV, 8), ((0, 0), (0, 8)))
    sb = jnp.concatenate([jnp.tile(action_scale, 2), jnp.tile(action_bias, 2)])
    out = _sc_gather(flat_idx, tab2, sb)
    return out.reshape(B, 8)


# speedup vs baseline: 3.3109x; 2.5953x over previous
import functools

import jax
import jax.numpy as jnp
from jax import lax
from jax.experimental import pallas as pl
from jax.experimental.pallas import tpu as pltpu
from jax.experimental.pallas import tpu_sc as plsc

B = 4096
OBS = 64
BITS = 12
NK = BITS + 1
SIZE = 512
NR2 = 43
NV = 4096

NC = 2
NS = 16
NW = NC * NS
BPW = B // NW
IPW = BPW * NR2
CHUNK = 128
NCHUNK = IPW // CHUNK


NBB = 4
BB = B // NBB


def _addr_body(x_ref, thr_ref, tab_ref, out_ref, w_ref):
    k = pl.program_id(0)
    b = pl.program_id(1)

    @pl.when(b == 0)
    def _():
        off = jnp.bitwise_and(lax.shift_left(1, k) - 1, 7)
        w_ref[k, :, :] = tab_ref[:, pl.ds(off, 1), :][:, 0, :]

    @pl.when(k == NK - 1)
    def _():
        x = x_ref[...]
        thr = thr_ref[...]
        c = jnp.zeros((BB, OBS), jnp.int32)
        for t in range(BITS):
            c = c + (x > thr[:, t][None, :]).astype(jnp.int32)
        e = jnp.concatenate(
            [(c == kk).astype(jnp.float32) for kk in range(NK)], axis=1)
        w = w_ref[...].reshape(NK * OBS, SIZE)
        h1 = jnp.dot(e, w, precision=lax.Precision.HIGHEST,
                     preferred_element_type=jnp.float32)
        bits = (h1 > 0).astype(jnp.float32)
        ri = lax.broadcasted_iota(jnp.int32, (SIZE, NR2), 0)
        cj = lax.broadcasted_iota(jnp.int32, (SIZE, NR2), 1)
        pw = jnp.where(ri // BITS == cj,
                       jnp.left_shift(1, ri % BITS), 0).astype(jnp.float32)
        af = jnp.dot(bits, pw, precision=lax.Precision.HIGHEST,
                     preferred_element_type=jnp.float32)
        ramj = lax.broadcasted_iota(jnp.int32, (BB, NR2), 1)
        out_ref[...] = af.astype(jnp.int32) + NV * ramj


def _layer2_addresses(x, thresholds, table1):
    return pl.pallas_call(
        _addr_body,
        grid=(NK, NBB),
        in_specs=[
            pl.BlockSpec((BB, OBS), lambda k, b: (b, 0)),
            pl.BlockSpec((OBS, BITS), lambda k, b: (0, 0)),
            pl.BlockSpec((OBS, 8, SIZE),
                         lambda k, b: (0, lax.shift_right_logical(
                             lax.shift_left(1, k) - 1, 3), 0)),
        ],
        out_specs=pl.BlockSpec((BB, NR2), lambda k, b: (b, 0)),
        out_shape=jax.ShapeDtypeStruct((B, NR2), jnp.int32),
        scratch_shapes=[pltpu.VMEM((NK, OBS, SIZE), jnp.float32)],
    )(x, thresholds, table1)


def _sc_body(idx_hbm, tab_hbm, sb_hbm, out_hbm,
             idx_v, rows_v, out_v, sb_v, sem):
    wid = lax.axis_index("s") * NC + lax.axis_index("c")
    pltpu.sync_copy(idx_hbm.at[pl.ds(wid * IPW, IPW)], idx_v)
    pltpu.sync_copy(sb_hbm, sb_v)

    groups = [list(range(g, min(g + 11, NCHUNK))) for g in range(0, NCHUNK, 11)]
    copies = {}

    def fire(g):
        for c in groups[g]:
            copies[c] = pltpu.async_copy(
                tab_hbm.at[idx_v.at[pl.ds(c * CHUNK, CHUNK)]],
                rows_v.at[pl.ds(c * CHUNK, CHUNK), :], sem)

    def drain(g):
        for c in groups[g]:
            copies[c].wait()

    fire(0)
    for g in range(1, len(groups)):
        fire(g)
        drain(g - 1)
    drain(len(groups) - 1)

    iota = lax.iota(jnp.int32, 16)
    lo = jnp.bitwise_and(iota, 7)
    scale16 = sb_v[pl.ds(0, 16)]
    bias16 = sb_v[pl.ds(16, 16)]

    def reduce_one(bl):
        acc = jnp.zeros((16,), jnp.float32)
        for j in range(NR2):
            acc = acc + rows_v[bl * NR2 + j, :]
        return acc

    def pair(p, _):
        acc_a = reduce_one(2 * p)
        acc_b = reduce_one(2 * p + 1)
        perm_b = lax.gather(
            acc_b, lo[:, None],
            lax.GatherDimensionNumbers(offset_dims=(), collapsed_slice_dims=(0,),
                                       start_index_map=(0,)),
            slice_sizes=(1,), mode=lax.GatherScatterMode.PROMISE_IN_BOUNDS)
        acc = jnp.where(iota < 8, acc_a, perm_b)
        e = jnp.exp(acc + acc)
        th = 1.0 - 2.0 / (e + 1.0)
        out_v[pl.ds(p * 16, 16)] = th * scale16 + bias16
        return _

    lax.fori_loop(0, BPW // 2, pair, None)
    pltpu.sync_copy(out_v, out_hbm.at[pl.ds(wid * BPW * 8, BPW * 8)])


def _sc_gather(flat_idx, tab2, sb):
    call = functools.partial(
        pl.kernel,
        out_type=jax.ShapeDtypeStruct((B * 8,), jnp.float32),
        mesh=plsc.VectorSubcoreMesh(
            core_axis_name="c", subcore_axis_name="s", num_cores=NC),
        compiler_params=pltpu.CompilerParams(use_tc_tiling_on_sc=False),
        scratch_types=[
            pltpu.VMEM((IPW,), jnp.int32),
            pltpu.VMEM((IPW, 16), jnp.float32),
            pltpu.VMEM((BPW * 8,), jnp.float32),
            pltpu.VMEM((32,), jnp.float32),
            pltpu.SemaphoreType.DMA,
        ],
    )(_sc_body)
    return call(flat_idx, tab2, sb)


def kernel(x, thresholds, table1, table2, action_scale, action_bias):
    flat_idx = _layer2_addresses(x, thresholds, table1).reshape(-1)
    tab2 = jnp.pad(table2.reshape(NR2 * NV, 8), ((0, 0), (0, 8)))
    sb = jnp.concatenate([jnp.tile(action_scale, 2), jnp.tile(action_bias, 2)])
    out = _sc_gather(flat_idx, tab2, sb)
    return out.reshape(B, 8)


# speedup vs baseline: 3.9303x; 1.1871x over previous
import functools

import jax
import jax.numpy as jnp
from jax import lax
from jax.experimental import pallas as pl
from jax.experimental.pallas import tpu as pltpu
from jax.experimental.pallas import tpu_sc as plsc

B = 4096
OBS = 64
BITS = 12
NK = BITS + 1
SIZE = 512
NR2 = 43
NV = 4096

NC = 2
NS = 16
NW = NC * NS
BPW = B // NW
IPW = BPW * NR2
CHUNK = 128
NCHUNK = IPW // CHUNK


NBB = 4
BB = B // NBB


def _addr_body(x_ref, thr_ref, tab_ref, out_ref, w_ref):
    k = pl.program_id(0)
    b = pl.program_id(1)

    @pl.when(b == 0)
    def _():
        off = jnp.bitwise_and(lax.shift_left(1, k) - 1, 7)
        w_ref[k, :, :] = tab_ref[:, pl.ds(off, 1), :][:, 0, :]

    @pl.when(k == NK - 1)
    def _():
        x = x_ref[...]
        thr = thr_ref[...]
        c = jnp.zeros((BB, OBS), jnp.int32)
        for t in range(BITS):
            c = c + (x > thr[:, t][None, :]).astype(jnp.int32)
        e = jnp.concatenate(
            [(c == kk).astype(jnp.float32) for kk in range(NK)], axis=1)
        w = w_ref[...].reshape(NK * OBS, SIZE)
        h1 = jnp.dot(e, w, precision=lax.Precision.HIGHEST,
                     preferred_element_type=jnp.float32)
        bits = (h1 > 0).astype(jnp.float32)
        ri = lax.broadcasted_iota(jnp.int32, (SIZE, NR2), 0)
        cj = lax.broadcasted_iota(jnp.int32, (SIZE, NR2), 1)
        pw = jnp.where(ri // BITS == cj,
                       jnp.left_shift(1, ri % BITS), 0).astype(jnp.float32)
        af = jnp.dot(bits, pw, precision=lax.Precision.HIGHEST,
                     preferred_element_type=jnp.float32)
        ramj = lax.broadcasted_iota(jnp.int32, (BB, NR2), 1)
        addr = af.astype(jnp.int32) + NV * ramj
        out_ref[...] = jnp.concatenate(
            [addr, jnp.zeros((BB, 1), jnp.int32)], axis=1)


def _layer2_addresses(x, thresholds, table1):
    return pl.pallas_call(
        _addr_body,
        grid=(NK, NBB),
        in_specs=[
            pl.BlockSpec((BB, OBS), lambda k, b: (b, 0)),
            pl.BlockSpec((OBS, BITS), lambda k, b: (0, 0)),
            pl.BlockSpec((OBS, 8, SIZE),
                         lambda k, b: (0, lax.shift_right_logical(
                             lax.shift_left(1, k) - 1, 3), 0)),
        ],
        out_specs=pl.BlockSpec((BB, NR2 + 1), lambda k, b: (b, 0)),
        out_shape=jax.ShapeDtypeStruct((B, NR2 + 1), jnp.int32),
        scratch_shapes=[pltpu.VMEM((NK, OBS, SIZE), jnp.float32)],
    )(x, thresholds, table1)


NPAIR = BPW // 2
PROW = 2 * (NR2 + 1)
EVREG = (NR2 + 1) // 2


def _sc_body(idx_hbm, tab_hbm, sb_hbm, out_hbm,
             idx_v, rows_v, out_v, sb_v, sem):
    wid = lax.axis_index("s") * NC + lax.axis_index("c")
    pltpu.sync_copy(idx_hbm.at[pl.ds(wid * BPW, BPW), :], idx_v)
    pltpu.sync_copy(sb_hbm, sb_v)
    tab2d = tab_hbm

    groups = [list(range(g, min(g + 16, BPW))) for g in range(0, BPW, 16)]
    copies = {}

    def fire(g):
        for e in groups[g]:
            copies[e] = pltpu.async_copy(
                tab2d.at[idx_v.at[e]],
                rows_v.at[pl.ds(e * (NR2 + 1), NR2 + 1), :], sem)

    def drain(g):
        for p in groups[g]:
            copies[p].wait()

    fire(0)
    for g in range(1, len(groups)):
        fire(g)
        drain(g - 1)
    drain(len(groups) - 1)

    iota = lax.iota(jnp.int32, 16)
    scale16 = sb_v[pl.ds(0, 16)]
    bias16 = sb_v[pl.ds(16, 16)]
    col16 = jnp.bitwise_and(iota, 7)
    half = jnp.right_shift(iota, 3)

    def pair(p, _):
        rowb = (2 * p + half) * (NR2 + 1)
        acc = jnp.zeros((16,), jnp.float32)
        for j in range(NR2):
            acc = acc + plsc.load_gather(rows_v, [rowb + j, col16])
        e = jnp.exp(acc + acc)
        th = 1.0 - 2.0 / (e + 1.0)
        out_v[pl.ds(p * 16, 16)] = th * scale16 + bias16
        return _

    lax.fori_loop(0, NPAIR, pair, None)
    pltpu.sync_copy(out_v, out_hbm.at[pl.ds(wid * BPW * 8, BPW * 8)])


def _sc_gather(flat_idx, tab2, sb):
    call = functools.partial(
        pl.kernel,
        out_type=jax.ShapeDtypeStruct((B * 8,), jnp.float32),
        mesh=plsc.VectorSubcoreMesh(
            core_axis_name="c", subcore_axis_name="s", num_cores=NC),
        compiler_params=pltpu.CompilerParams(
            use_tc_tiling_on_sc=False, needs_layout_passes=False),
        scratch_types=[
            pltpu.VMEM((BPW, NR2 + 1), jnp.int32),
            pltpu.VMEM((NPAIR * PROW, 8), jnp.float32),
            pltpu.VMEM((BPW * 8,), jnp.float32),
            pltpu.VMEM((32,), jnp.float32),
            pltpu.SemaphoreType.DMA,
        ],
    )(_sc_body)
    return call(flat_idx, tab2, sb)


def kernel(x, thresholds, table1, table2, action_scale, action_bias):
    idx = _layer2_addresses(x, thresholds, table1)
    sb = jnp.concatenate([jnp.tile(action_scale, 2), jnp.tile(action_bias, 2)])
    out = _sc_gather(idx, table2.reshape(NR2 * NV, 8), sb)
    return out.reshape(B, 8)


# speedup vs baseline: 4.2948x; 1.0928x over previous
import functools

import jax
import jax.numpy as jnp
from jax import lax
from jax.experimental import pallas as pl
from jax.experimental.pallas import tpu as pltpu
from jax.experimental.pallas import tpu_sc as plsc

B = 4096
OBS = 64
BITS = 12
NK = BITS + 1
SIZE = 512
NR2 = 43
NV = 4096

NC = 2
NS = 16
NW = NC * NS
BPW = B // NW

NBB = 4
BB = B // NBB
WPB = BB // BPW


def _addr_body(x_ref, thr_ref, tab_ref, out_ref, w_ref):
    k = pl.program_id(0)
    b = pl.program_id(1)

    @pl.when(b == 0)
    def _():
        off = jnp.bitwise_and(lax.shift_left(1, k) - 1, 7)
        w_ref[k, :, :] = tab_ref[:, pl.ds(off, 1), :][:, 0, :]

    @pl.when(k == NK - 1)
    def _():
        x = x_ref[...]
        thr = thr_ref[...]
        c = jnp.zeros((BB, OBS), jnp.int32)
        for t in range(BITS):
            c = c + (x > thr[:, t][None, :]).astype(jnp.int32)
        e = jnp.concatenate(
            [(c == kk).astype(jnp.float32) for kk in range(NK)], axis=1)
        w = w_ref[...].reshape(NK * OBS, SIZE)
        h1 = jnp.dot(e, w, precision=lax.Precision.HIGHEST,
                     preferred_element_type=jnp.float32)
        bits = (h1 > 0).astype(jnp.float32)
        ri = lax.broadcasted_iota(jnp.int32, (SIZE, NR2), 0)
        cj = lax.broadcasted_iota(jnp.int32, (SIZE, NR2), 1)
        pw = jnp.where(ri // BITS == cj,
                       jnp.left_shift(1, ri % BITS), 0).astype(jnp.float32)
        af = jnp.dot(bits, pw, precision=lax.Precision.HIGHEST,
                     preferred_element_type=jnp.float32)
        ramj = lax.broadcasted_iota(jnp.int32, (BB, NR2), 1)
        addr = af.astype(jnp.int32) + NV * ramj
        out_ref[...] = addr.reshape(WPB, BPW, NR2).transpose(0, 2, 1)


def _layer2_addresses(x, thresholds, table1):
    return pl.pallas_call(
        _addr_body,
        grid=(NK, NBB),
        in_specs=[
            pl.BlockSpec((BB, OBS), lambda k, b: (b, 0)),
            pl.BlockSpec((OBS, BITS), lambda k, b: (0, 0)),
            pl.BlockSpec((OBS, 8, SIZE),
                         lambda k, b: (0, lax.shift_right_logical(
                             lax.shift_left(1, k) - 1, 3), 0)),
        ],
        out_specs=pl.BlockSpec((WPB, NR2, BPW), lambda k, b: (b, 0, 0)),
        out_shape=jax.ShapeDtypeStruct((NW, NR2, BPW), jnp.int32),
        scratch_shapes=[pltpu.VMEM((NK, OBS, SIZE), jnp.float32)],
    )(x, thresholds, table1)


def _sc_body(idx_hbm, tab_hbm, sb_hbm, out_hbm,
             idx_v, rows_v, out_v, sb_v, sem):
    wid = lax.axis_index("s") * NC + lax.axis_index("c")
    pltpu.sync_copy(idx_hbm.at[wid], idx_v)
    pltpu.sync_copy(sb_hbm, sb_v)

    groups = [list(range(g, min(g + 11, NR2))) for g in range(0, NR2, 11)]
    copies = {}

    def fire(g):
        for j in groups[g]:
            copies[j] = pltpu.async_copy(
                tab_hbm.at[idx_v.at[j]],
                rows_v.at[pl.ds(j * BPW, BPW), :], sem)

    def drain(g):
        for j in groups[g]:
            copies[j].wait()

    fire(0)
    for g in range(1, len(groups)):
        fire(g)
        drain(g - 1)
    drain(len(groups) - 1)

    iota = lax.iota(jnp.int32, 16)
    scale16 = sb_v[pl.ds(0, 16)]
    bias16 = sb_v[pl.ds(16, 16)]
    col16 = jnp.bitwise_and(iota, 7)
    half = jnp.right_shift(iota, 3)

    def pair(p, _):
        e16 = 2 * p + half
        acc = jnp.zeros((16,), jnp.float32)
        for j in range(NR2):
            acc = acc + plsc.load_gather(rows_v, [j * BPW + e16, col16])
        e = jnp.exp(acc + acc)
        th = 1.0 - 2.0 / (e + 1.0)
        out_v[pl.ds(p * 16, 16)] = th * scale16 + bias16
        return _

    lax.fori_loop(0, BPW // 2, pair, None)
    pltpu.sync_copy(out_v, out_hbm.at[pl.ds(wid * BPW * 8, BPW * 8)])


def _sc_gather(idx, tab2, sb):
    call = functools.partial(
        pl.kernel,
        out_type=jax.ShapeDtypeStruct((B * 8,), jnp.float32),
        mesh=plsc.VectorSubcoreMesh(
            core_axis_name="c", subcore_axis_name="s", num_cores=NC),
        compiler_params=pltpu.CompilerParams(
            use_tc_tiling_on_sc=False, needs_layout_passes=False),
        scratch_types=[
            pltpu.VMEM((NR2, BPW), jnp.int32),
            pltpu.VMEM((NR2 * BPW, 8), jnp.float32),
            pltpu.VMEM((BPW * 8,), jnp.float32),
            pltpu.VMEM((32,), jnp.float32),
            pltpu.SemaphoreType.DMA,
        ],
    )(_sc_body)
    return call(idx, tab2, sb)


def kernel(x, thresholds, table1, table2, action_scale, action_bias):
    idx = _layer2_addresses(x, thresholds, table1)
    sb = jnp.concatenate([jnp.tile(action_scale, 2), jnp.tile(action_bias, 2)])
    out = _sc_gather(idx, table2.reshape(NR2 * NV, 8), sb)
    return out.reshape(B, 8)


# speedup vs baseline: 5.2982x; 1.2336x over previous
import functools

import jax
import jax.numpy as jnp
from jax import lax
from jax.experimental import pallas as pl
from jax.experimental.pallas import tpu as pltpu
from jax.experimental.pallas import tpu_sc as plsc

B = 4096
OBS = 64
BITS = 12
NK = BITS + 1
SIZE = 512
NR2 = 43
NV = 4096

NC = 2
NS = 16
NW = NC * NS
BPW = B // NW

NBB = 4
BB = B // NBB
WPB = BB // BPW


def _addr_body(x_ref, thr_ref, tab_ref, out_ref, w_ref, sem):
    b = pl.program_id(0)

    @pl.when(b == 0)
    def _():
        cps = [pltpu.async_copy(tab_ref.at[:, 2 ** k - 1, :], w_ref.at[k], sem)
               for k in range(NK)]
        for cp in cps:
            cp.wait()

    x = x_ref[...]
    thr = thr_ref[...]
    c = jnp.zeros((BB, OBS), jnp.int32)
    for t in range(BITS):
        c = c + (x > thr[:, t][None, :]).astype(jnp.int32)
    e = jnp.concatenate(
        [(c == kk).astype(jnp.float32) for kk in range(NK)], axis=1)
    w = w_ref[...].reshape(NK * OBS, SIZE)
    h1 = jnp.dot(e, w, precision=lax.Precision.HIGHEST,
                 preferred_element_type=jnp.float32)
    bits = (h1 > 0).astype(jnp.float32)
    ri = lax.broadcasted_iota(jnp.int32, (SIZE, NR2), 0)
    cj = lax.broadcasted_iota(jnp.int32, (SIZE, NR2), 1)
    pw = jnp.where(ri // BITS == cj,
                   jnp.left_shift(1, ri % BITS), 0).astype(jnp.float32)
    af = jnp.dot(bits, pw, precision=lax.Precision.HIGHEST,
                 preferred_element_type=jnp.float32)
    addr = af.astype(jnp.int32)
    out_ref[...] = addr.reshape(WPB, BPW, NR2).transpose(0, 2, 1)


def _layer2_addresses(x, thresholds, table1):
    return pl.pallas_call(
        _addr_body,
        grid=(NBB,),
        in_specs=[
            pl.BlockSpec((BB, OBS), lambda b: (b, 0)),
            pl.BlockSpec((OBS, BITS), lambda b: (0, 0)),
            pl.BlockSpec(memory_space=pl.ANY),
        ],
        out_specs=pl.BlockSpec((WPB, NR2, BPW), lambda b: (b, 0, 0)),
        out_shape=jax.ShapeDtypeStruct((NW, NR2, BPW), jnp.int32),
        scratch_shapes=[pltpu.VMEM((NK, OBS, SIZE), jnp.float32),
                        pltpu.SemaphoreType.DMA],
    )(x, thresholds, table1)


def _sc_body(idx_hbm, tab_hbm, sb_hbm, out_hbm,
             idx_v, rows_v, out_v, sb_v, sem):
    wid = lax.axis_index("s") * NC + lax.axis_index("c")
    pltpu.sync_copy(idx_hbm.at[wid], idx_v)
    pltpu.sync_copy(sb_hbm, sb_v)

    groups = [list(range(g, min(g + 11, NR2))) for g in range(0, NR2, 11)]
    copies = {}

    def fire(g):
        for j in groups[g]:
            copies[j] = pltpu.async_copy(
                tab_hbm.at[j].at[idx_v.at[j]],
                rows_v.at[pl.ds(j * BPW, BPW), :], sem)

    def drain(g):
        for j in groups[g]:
            copies[j].wait()

    fire(0)
    for g in range(1, len(groups)):
        fire(g)
        drain(g - 1)
    drain(len(groups) - 1)

    iota = lax.iota(jnp.int32, 16)
    scale16 = sb_v[pl.ds(0, 16)]
    bias16 = sb_v[pl.ds(16, 16)]
    col16 = jnp.bitwise_and(iota, 7)
    half = jnp.right_shift(iota, 3)

    def pair(p, _):
        e16 = 2 * p + half
        acc = jnp.zeros((16,), jnp.float32)
        for j in range(NR2):
            acc = acc + plsc.load_gather(rows_v, [j * BPW + e16, col16])
        e = jnp.exp(acc + acc)
        th = 1.0 - 2.0 / (e + 1.0)
        out_v[pl.ds(p * 16, 16)] = th * scale16 + bias16
        return _

    lax.fori_loop(0, BPW // 2, pair, None)
    pltpu.sync_copy(out_v, out_hbm.at[pl.ds(wid * BPW * 8, BPW * 8)])


def _sc_gather(idx, tab2, sb):
    call = functools.partial(
        pl.kernel,
        out_type=jax.ShapeDtypeStruct((B * 8,), jnp.float32),
        mesh=plsc.VectorSubcoreMesh(
            core_axis_name="c", subcore_axis_name="s", num_cores=NC),
        compiler_params=pltpu.CompilerParams(
            use_tc_tiling_on_sc=False, needs_layout_passes=False),
        scratch_types=[
            pltpu.VMEM((NR2, BPW), jnp.int32),
            pltpu.VMEM((NR2 * BPW, 8), jnp.float32),
            pltpu.VMEM((BPW * 8,), jnp.float32),
            pltpu.VMEM((32,), jnp.float32),
            pltpu.SemaphoreType.DMA,
        ],
    )(_sc_body)
    return call(idx, tab2, sb)


def kernel(x, thresholds, table1, table2, action_scale, action_bias):
    idx = _layer2_addresses(x, thresholds, table1)
    sb = jnp.concatenate([jnp.tile(action_scale, 2), jnp.tile(action_bias, 2)])
    out = _sc_gather(idx, table2, sb)
    return out.reshape(B, 8)


# speedup vs baseline: 5.9415x; 1.1214x over previous
import functools

import jax
import jax.numpy as jnp
from jax import lax
from jax.experimental import pallas as pl
from jax.experimental.pallas import tpu as pltpu
from jax.experimental.pallas import tpu_sc as plsc

B = 4096
OBS = 64
BITS = 12
NK = BITS + 1
SIZE = 512
NR2 = 43
NV = 4096

NC = 2
NS = 16
NW = NC * NS
BPW = B // NW

NBB = 4
BB = B // NBB
WPB = BB // BPW


def _addr_body(x_ref, thr_ref, tab_ref, out_ref, w_ref, sem):
    b = pl.program_id(0)

    @pl.when(b == 0)
    def _():
        cps = [pltpu.async_copy(tab_ref.at[:, 2 ** k - 1, :], w_ref.at[k], sem)
               for k in range(NK)]
        for cp in cps:
            cp.wait()

    x = x_ref[...]
    thr = thr_ref[...]
    c = jnp.zeros((BB, OBS), jnp.int32)
    for t in range(BITS):
        c = c + (x > thr[:, t][None, :]).astype(jnp.int32)
    e = jnp.concatenate(
        [(c == kk).astype(jnp.bfloat16) for kk in range(NK)], axis=1)
    w = w_ref[...].reshape(NK * OBS, SIZE)
    w_hi = w.astype(jnp.bfloat16)
    w_lo = (w - w_hi.astype(jnp.float32)).astype(jnp.bfloat16)
    h1 = (jnp.dot(e, w_hi, preferred_element_type=jnp.float32)
          + jnp.dot(e, w_lo, preferred_element_type=jnp.float32))
    bits = (h1 > 0).astype(jnp.float32)
    ri = lax.broadcasted_iota(jnp.int32, (SIZE, NR2), 0)
    cj = lax.broadcasted_iota(jnp.int32, (SIZE, NR2), 1)
    pw = jnp.where(ri // BITS == cj,
                   jnp.left_shift(1, ri % BITS), 0).astype(jnp.float32)
    af = jnp.dot(bits, pw, precision=lax.Precision.HIGHEST,
                 preferred_element_type=jnp.float32)
    addr = af.astype(jnp.int32)
    out_ref[...] = addr.reshape(WPB, BPW, NR2).transpose(0, 2, 1)


def _layer2_addresses(x, thresholds, table1):
    return pl.pallas_call(
        _addr_body,
        grid=(NBB,),
        in_specs=[
            pl.BlockSpec((BB, OBS), lambda b: (b, 0)),
            pl.BlockSpec((OBS, BITS), lambda b: (0, 0)),
            pl.BlockSpec(memory_space=pl.ANY),
        ],
        out_specs=pl.BlockSpec((WPB, NR2, BPW), lambda b: (b, 0, 0)),
        out_shape=jax.ShapeDtypeStruct((NW, NR2, BPW), jnp.int32),
        scratch_shapes=[pltpu.VMEM((NK, OBS, SIZE), jnp.float32),
                        pltpu.SemaphoreType.DMA],
    )(x, thresholds, table1)


def _sc_body(idx_hbm, tab_hbm, sb_hbm, out_hbm,
             idx_v, rows_v, out_v, sb_v, sem):
    wid = lax.axis_index("s") * NC + lax.axis_index("c")
    pltpu.sync_copy(idx_hbm.at[wid], idx_v)
    pltpu.sync_copy(sb_hbm, sb_v)

    groups = [list(range(g, min(g + 11, NR2))) for g in range(0, NR2, 11)]
    copies = {}

    def fire(g):
        for j in groups[g]:
            copies[j] = pltpu.async_copy(
                tab_hbm.at[j].at[idx_v.at[j]],
                rows_v.at[pl.ds(j * BPW, BPW), :], sem)

    def drain(g):
        for j in groups[g]:
            copies[j].wait()

    fire(0)
    for g in range(1, len(groups)):
        fire(g)
        drain(g - 1)
    drain(len(groups) - 1)

    iota = lax.iota(jnp.int32, 16)
    scale16 = sb_v[pl.ds(0, 16)]
    bias16 = sb_v[pl.ds(16, 16)]
    col16 = jnp.bitwise_and(iota, 7)
    half = jnp.right_shift(iota, 3)

    def pair(p, _):
        e16 = 2 * p + half
        acc = jnp.zeros((16,), jnp.float32)
        for j in range(NR2):
            acc = acc + plsc.load_gather(rows_v, [j * BPW + e16, col16])
        e = jnp.exp(acc + acc)
        th = 1.0 - 2.0 / (e + 1.0)
        out_v[pl.ds(p * 16, 16)] = th * scale16 + bias16
        return _

    lax.fori_loop(0, BPW // 2, pair, None)
    pltpu.sync_copy(out_v, out_hbm.at[pl.ds(wid * BPW * 8, BPW * 8)])


def _sc_gather(idx, tab2, sb):
    call = functools.partial(
        pl.kernel,
        out_type=jax.ShapeDtypeStruct((B * 8,), jnp.float32),
        mesh=plsc.VectorSubcoreMesh(
            core_axis_name="c", subcore_axis_name="s", num_cores=NC),
        compiler_params=pltpu.CompilerParams(
            use_tc_tiling_on_sc=False, needs_layout_passes=False),
        scratch_types=[
            pltpu.VMEM((NR2, BPW), jnp.int32),
            pltpu.VMEM((NR2 * BPW, 8), jnp.float32),
            pltpu.VMEM((BPW * 8,), jnp.float32),
            pltpu.VMEM((32,), jnp.float32),
            pltpu.SemaphoreType.DMA,
        ],
    )(_sc_body)
    return call(idx, tab2, sb)


def kernel(x, thresholds, table1, table2, action_scale, action_bias):
    idx = _layer2_addresses(x, thresholds, table1)
    sb = jnp.concatenate([jnp.tile(action_scale, 2), jnp.tile(action_bias, 2)])
    out = _sc_gather(idx, table2, sb)
    return out.reshape(B, 8)
